# Initial kernel scaffold; baseline (speedup 1.0000x reference)
import jax
import jax.numpy as jnp
from jax.experimental import pallas as pl


def kernel(node_feat, edge_index, edge_feat, V1, w_comp1, loop1, b1, V2, w_comp2, loop2, b2):
    raise NotImplementedError("write your pallas kernel here")


# speedup vs baseline: 4.7297x; 4.7297x over previous
import functools

import jax
import jax.numpy as jnp
from jax import lax
from jax.experimental import pallas as pl
from jax.experimental.pallas import tpu as pltpu
from jax.experimental.pallas import tpu_sc as plsc

N = 10000
NP = 10240
E = 320000
F = 128
NC = 2
NS = 16
NW = NC * NS
EPW = E // NW
K1 = 128
NCH = EPW // K1
KT = EPW - NCH * K1
RPS = NP // NS
ZR = 128


def _sc_agg_body(x_hbm, src_hbm, dst_hbm, feat_hbm, out_hbm,
                 acc, zbuf, src_v, dst_v, feat_v, rows_v,
                 src_t, dst_t, feat_t, rows_t, sem):
    c = lax.axis_index("c")
    s = lax.axis_index("s")
    wid = c * NS + s

    def zero_row(i, carry):
        for cc in range(F // 16):
            zbuf[i, pl.ds(cc * 16, 16)] = jnp.zeros((16,), jnp.float32)
        return carry
    lax.fori_loop(0, ZR, zero_row, 0)
    for r in range(RPS // ZR):
        pltpu.sync_copy(zbuf, acc.at[pl.ds(s * RPS + r * ZR, ZR)])
    plsc.subcore_barrier()

    def do_chunk(base, ksz, srcr, dstr, featr, rowsr):
        base = pl.multiple_of(base, 8)
        pltpu.sync_copy(src_hbm.at[pl.ds(base, ksz)], srcr)
        pltpu.sync_copy(dst_hbm.at[pl.ds(base, ksz)], dstr)
        pltpu.sync_copy(feat_hbm.at[pl.ds(base, ksz)], featr)
        pltpu.async_copy(x_hbm.at[srcr], rowsr, sem).wait()

        dnums = lax.GatherDimensionNumbers(
            offset_dims=(), collapsed_slice_dims=(0,), start_index_map=(0,))

        def edge_group(g, carry):
            fvec = featr[pl.ds(g * 16, 16)]
            for jj in range(16):
                f = lax.gather(
                    fvec, jnp.full((16, 1), jj, jnp.int32), dnums, (1,),
                    mode=lax.GatherScatterMode.PROMISE_IN_BOUNDS)
                j = g * 16 + jj
                for cc in range(F // 16):
                    sl = pl.ds(cc * 16, 16)
                    rowsr[j, sl] = rowsr[j, sl] * f
            return carry
        lax.fori_loop(0, ksz // 16, edge_group, 0)
        pltpu.sync_copy(rowsr, acc.at[dstr], add=True)

    def chunk(i, carry):
        do_chunk(wid * EPW + i * K1, K1, src_v, dst_v, feat_v, rows_v)
        return carry
    lax.fori_loop(0, NCH, chunk, 0)
    do_chunk(wid * EPW + NCH * K1, KT, src_t, dst_t, feat_t, rows_t)

    plsc.subcore_barrier()
    for r in range(RPS // ZR):
        rows = pl.ds(s * RPS + r * ZR, ZR)
        pltpu.sync_copy(acc.at[rows],
                        out_hbm.at[pl.ds(c * NP + s * RPS + r * ZR, ZR)])


_sc_agg = pl.kernel(
    _sc_agg_body,
    mesh=plsc.VectorSubcoreMesh(core_axis_name="c", subcore_axis_name="s"),
    out_type=jax.ShapeDtypeStruct((NC * NP, F), jnp.float32),
    scratch_types=[
        pltpu.VMEM_SHARED((NP, F), jnp.float32),
        pltpu.VMEM((ZR, F), jnp.float32),
        pltpu.VMEM((K1,), jnp.int32),
        pltpu.VMEM((K1,), jnp.int32),
        pltpu.VMEM((K1,), jnp.float32),
        pltpu.VMEM((K1, F), jnp.float32),
        pltpu.VMEM((KT,), jnp.int32),
        pltpu.VMEM((KT,), jnp.int32),
        pltpu.VMEM((KT,), jnp.float32),
        pltpu.VMEM((KT, F), jnp.float32),
        pltpu.SemaphoreType.DMA,
    ],
)


ROWS_BLK = 1000


def _combine_body(relu, parts_ref, x_ref, v_ref, w_ref, loop_ref, b_ref,
                  out_ref, w0_ref):
    @pl.when(pl.program_id(0) == 0)
    def _():
        acc = w_ref[0, 0] * v_ref[0]
        for b in range(1, v_ref.shape[0]):
            acc = acc + w_ref[0, b] * v_ref[b]
        w0_ref[...] = acc

    a = parts_ref[0] + parts_ref[1]
    y = jnp.dot(a, w0_ref[...], preferred_element_type=jnp.float32)
    y = y + jnp.dot(x_ref[...], loop_ref[...],
                    preferred_element_type=jnp.float32)
    y = y + b_ref[...]
    out_ref[...] = jnp.maximum(y, 0.0) if relu else y


def _combine(parts, x, v, w_comp, loop_w, b, relu):
    nb = v.shape[0]
    return pl.pallas_call(
        functools.partial(_combine_body, relu),
        grid=(N // ROWS_BLK,),
        in_specs=[
            pl.BlockSpec((2, ROWS_BLK, F), lambda i: (0, i, 0)),
            pl.BlockSpec((ROWS_BLK, F), lambda i: (i, 0)),
            pl.BlockSpec((nb, F, F), lambda i: (0, 0, 0)),
            pl.BlockSpec((1, nb), lambda i: (0, 0),
                         memory_space=pltpu.SMEM),
            pl.BlockSpec((F, F), lambda i: (0, 0)),
            pl.BlockSpec((1, F), lambda i: (0, 0)),
        ],
        out_specs=pl.BlockSpec((ROWS_BLK, F), lambda i: (i, 0)),
        out_shape=jax.ShapeDtypeStruct((N, F), jnp.float32),
        scratch_shapes=[pltpu.VMEM((F, F), jnp.float32)],
    )(parts.reshape(NC, NP, F), x, v, w_comp[:1], loop_w, b.reshape(1, F))


def kernel(node_feat, edge_index, edge_feat, V1, w_comp1, loop1, b1,
           V2, w_comp2, loop2, b2):
    src = edge_index[0]
    dst = edge_index[1]
    feat = edge_feat[:, 0]
    parts1 = _sc_agg(node_feat, src, dst, feat)
    h = _combine(parts1, node_feat, V1, w_comp1, loop1, b1, relu=True)
    parts2 = _sc_agg(h, src, dst, feat)
    return _combine(parts2, h, V2, w_comp2, loop2, b2, relu=False)
